# baseline (speedup 1.0000x reference)
import functools

import jax
import jax.numpy as jnp
from jax import lax
from jax.experimental import pallas as pl
from jax.experimental.pallas import tpu as pltpu
from jax.experimental.pallas import tpu_sc as plsc

D = 64
DP = 128
B = 4096 * 50
NC = 2
NS = 16
NW = NC * NS
BPW = B // NW
CHUNK = 64
K = 5
ROWS = K * CHUNK
STEPS = BPW // ROWS

_mesh = plsc.VectorSubcoreMesh(core_axis_name="c", subcore_axis_name="s")


@functools.partial(
    pl.kernel,
    mesh=_mesh,
    compiler_params=pltpu.CompilerParams(use_tc_tiling_on_sc=False),
    out_type=jax.ShapeDtypeStruct((B, D), jnp.float32),
    scratch_types=[
        pltpu.VMEM((STEPS * K, CHUNK), jnp.int32),
        pltpu.VMEM((ROWS, DP), jnp.float32),
        pltpu.VMEM((ROWS, DP), jnp.float32),
        pltpu.SemaphoreType.DMA,
        pltpu.SemaphoreType.DMA,
        pltpu.SemaphoreType.DMA,
        pltpu.SemaphoreType.DMA,
    ],
)
def _embed(idx_hbm, table_hbm, out_hbm, idx_v, rows0, rows1, g0, g1, w0, w1):
    wid = lax.axis_index("s") * NC + lax.axis_index("c")
    base = wid * BPW
    pltpu.sync_copy(idx_hbm.at[wid], idx_v)

    bufs = ((rows0, g0, w0), (rows1, g1, w1))

    def fire(t, rows, gsem):
        handles = []
        for j in range(K):
            handles.append(pltpu.async_copy(
                table_hbm.at[idx_v.at[t * K + j]],
                rows.at[pl.ds(j * CHUNK, CHUNK)],
                gsem,
            ))
        return handles

    def step2(s):
        all_handles = []
        for b, (rows, gsem, wsem) in enumerate(bufs):
            t = s + b

            @pl.when(t >= 2)
            def _():
                pltpu.make_async_copy(
                    rows.at[:, :D],
                    out_hbm.at[pl.ds(base + (t - 2) * ROWS, ROWS)],
                    wsem,
                ).wait()

            all_handles.append(fire(t, rows, gsem))

        for b, (rows, gsem, wsem) in enumerate(bufs):
            t = s + b
            for h in all_handles[b]:
                h.wait()
            pltpu.async_copy(
                rows.at[:, :D],
                out_hbm.at[pl.ds(base + t * ROWS, ROWS)],
                wsem,
            )

    pl.loop(0, STEPS, step=2)(step2)

    for b, (rows, gsem, wsem) in enumerate(bufs):
        t = STEPS - 2 + b
        pltpu.make_async_copy(
            rows.at[:, :D],
            out_hbm.at[pl.ds(base + t * ROWS, ROWS)],
            wsem,
        ).wait()


def kernel(x, table):
    idx = x.astype(jnp.int32).reshape(NW, STEPS * K, CHUNK)
    tpad = jnp.pad(table, ((0, 0), (0, DP - D)))
    out = _embed(idx, tpad)
    return out.reshape(x.shape[0], x.shape[1], D)
